# Initial kernel scaffold; baseline (speedup 1.0000x reference)
import jax
import jax.numpy as jnp
from jax.experimental import pallas as pl


def kernel(x, dtw_y):
    raise NotImplementedError("write your pallas kernel here")


# speedup vs baseline: 4.0963x; 4.0963x over previous
import functools

import jax
import jax.numpy as jnp
from jax import lax
from jax.experimental import pallas as pl
from jax.experimental.pallas import tpu as pltpu
from jax.experimental.pallas import tpu_sc as plsc

_INFO = plsc.get_sparse_core_info()
_NC = _INFO.num_cores
_NS = _INFO.num_subcores
_NW = _NC * _NS

_V, _D = 100000, 64
_B, _T = 4096, 50
_TOTAL = _B * _T
_G = 128
_ROWS_PER_W = _TOTAL // _NW
_NG = _ROWS_PER_W // _G


def _make_gather():
    mesh = plsc.VectorSubcoreMesh(core_axis_name="c", subcore_axis_name="s")

    @functools.partial(
        pl.kernel,
        out_type=jax.ShapeDtypeStruct((_TOTAL, _D), jnp.float32),
        mesh=mesh,
        scratch_types=[
            pltpu.VMEM((_NG, _G), jnp.int32),
            pltpu.VMEM((_G, _D), jnp.float32),
            pltpu.SemaphoreType.DMA,
        ],
        compiler_params=pltpu.CompilerParams(use_tc_tiling_on_sc=False),
    )
    def gather(x_hbm, idx_hbm, out_hbm, idx_v, buf, sem):
        wid = lax.axis_index("s") * _NC + lax.axis_index("c")
        base = wid * _ROWS_PER_W
        pltpu.sync_copy(idx_hbm.at[wid], idx_v)

        def step(j, carry):
            pltpu.async_copy(x_hbm.at[idx_v.at[j]], buf, sem).wait()
            pltpu.sync_copy(buf, out_hbm.at[pl.ds(base + j * _G, _G)])
            return carry

        lax.fori_loop(0, _NG, step, 0)

    return gather


_gather = _make_gather()


def kernel(x, dtw_y):
    idx = dtw_y.reshape(_NW, _NG, _G)
    out = _gather(x, idx)
    return out.reshape(_B, _T, _D)


# speedup vs baseline: 4.6039x; 1.1239x over previous
import functools

import jax
import jax.numpy as jnp
from jax import lax
from jax.experimental import pallas as pl
from jax.experimental.pallas import tpu as pltpu
from jax.experimental.pallas import tpu_sc as plsc

_INFO = plsc.get_sparse_core_info()
_NC = _INFO.num_cores
_NS = _INFO.num_subcores
_NW = _NC * _NS

_V, _D = 100000, 64
_B, _T = 4096, 50
_TOTAL = _B * _T
_G = 128
_ROWS_PER_W = _TOTAL // _NW
_NG = _ROWS_PER_W // _G


_K = 5
_BLK = _K * _G
_NBLK = _NG // _K


def _make_gather():
    mesh = plsc.VectorSubcoreMesh(core_axis_name="c", subcore_axis_name="s")

    @functools.partial(
        pl.kernel,
        out_type=jax.ShapeDtypeStruct((_TOTAL, _D), jnp.float32),
        mesh=mesh,
        scratch_types=[
            pltpu.VMEM((_NG, _G), jnp.int32),
            pltpu.VMEM((2, _BLK, _D), jnp.float32),
            pltpu.SemaphoreType.DMA,
            pltpu.SemaphoreType.DMA,
        ],
        compiler_params=pltpu.CompilerParams(use_tc_tiling_on_sc=False),
    )
    def gather(x_hbm, idx_hbm, out_hbm, idx_v, buf, gsem, ssem):
        wid = lax.axis_index("s") * _NC + lax.axis_index("c")
        base = wid * _ROWS_PER_W
        pltpu.sync_copy(idx_hbm.at[wid], idx_v)

        def fire_gathers(b, half):
            for k in range(_K):
                pltpu.async_copy(
                    x_hbm.at[idx_v.at[b * _K + k]],
                    buf.at[half, pl.ds(k * _G, _G)],
                    gsem,
                )

        def drain(ref, sem):
            pltpu.make_async_copy(out_hbm.at[pl.ds(0, _BLK)], ref, sem).wait()

        fire_gathers(0, 0)

        def step(b, carry):
            half = lax.rem(b, 2)
            prev = 1 - half
            drain(buf.at[prev], gsem)
            @pl.when(b >= 2)
            def _():
                drain(buf.at[half], ssem)
            fire_gathers(b, half)
            pltpu.async_copy(
                buf.at[prev],
                out_hbm.at[pl.ds(base + (b - 1) * _BLK, _BLK)],
                ssem,
            )
            return carry

        lax.fori_loop(1, _NBLK, step, 0)

        last = (_NBLK - 1) % 2
        drain(buf.at[last], gsem)
        pltpu.async_copy(
            buf.at[last],
            out_hbm.at[pl.ds(base + (_NBLK - 1) * _BLK, _BLK)],
            ssem,
        )
        drain(buf.at[0], ssem)
        drain(buf.at[1], ssem)

    return gather


_gather = _make_gather()


def kernel(x, dtw_y):
    idx = dtw_y.reshape(_NW, _NG, _G)
    out = _gather(x, idx)
    return out.reshape(_B, _T, _D)
